# Initial kernel scaffold; baseline (speedup 1.0000x reference)
import jax
import jax.numpy as jnp
from jax.experimental import pallas as pl


def kernel(x, edge_index, batch, gw0, gb0, gw1, gb1, gw2, gb2, gw3, gb3, c1w, c1b, c2w, c2b, mw1, mb1, mw2, mb2):
    raise NotImplementedError("write your pallas kernel here")


# speedup vs baseline: 12.0119x; 12.0119x over previous
import functools

import jax
import jax.numpy as jnp
from jax import lax
from jax.experimental import pallas as pl
from jax.experimental.pallas import tpu as pltpu
from jax.experimental.pallas import tpu_sc as plsc

N_NODES = 10000
N_EDGES = 320000
D_FEAT = 128
HID = 32
N_GRAPHS = 128
K_SORT = 30

NC = 2
NS = 16
NW = NC * NS
EW = 80
NWIN = N_EDGES // (NW * EW)

NPAD = 10240
NODE_WIN = 4
DP = 128
TRASH = N_GRAPHS * K_SORT
DENSE_PAD = 3968
NROWS = 10240

def _dot(a, b):
    return jnp.dot(a, b, preferred_element_type=jnp.float32)


def _sc_mesh():
    return plsc.VectorSubcoreMesh(core_axis_name="c", subcore_axis_name="s")


def _sc_deg(dst3, ones, zeros16):
    @functools.partial(
        pl.kernel,
        out_type=jax.ShapeDtypeStruct((NC, NROWS, DP), jnp.float32),
        mesh=_sc_mesh(),
        scratch_types=[
            pltpu.VMEM((NWIN, EW), jnp.int32),
            pltpu.VMEM((EW, DP), jnp.float32),
            pltpu.VMEM_SHARED((NROWS, DP), jnp.float32),
            pltpu.SemaphoreType.DMA,
        ],
    )
    def k(dst_hbm, ones_hbm, zeros_hbm, out_hbm, dst_v, ones_v, acc_sh, sem):
        cid = lax.axis_index("c")
        sid = lax.axis_index("s")
        wid = cid * NS + sid
        rows = NROWS // NS
        sl = pl.ds(sid * rows, rows)
        pltpu.async_copy(zeros_hbm.at[sl], acc_sh.at[sl], sem).wait()
        pltpu.async_copy(dst_hbm.at[wid], dst_v, sem).wait()
        pltpu.async_copy(ones_hbm, ones_v, sem).wait()
        plsc.subcore_barrier()

        @pl.loop(0, NWIN)
        def _(w):
            pltpu.sync_copy(ones_v, acc_sh.at[dst_v.at[w]], add=True)

        plsc.subcore_barrier()
        pltpu.async_copy(acc_sh.at[sl], out_hbm.at[cid, sl], sem).wait()

    return k(dst3, ones, zeros16)


def _sc_seg(g, src3, dst3, zeros):
    @functools.partial(
        pl.kernel,
        out_type=jax.ShapeDtypeStruct((NC, NROWS, DP), jnp.float32),
        mesh=_sc_mesh(),
        scratch_types=[
            pltpu.VMEM((NWIN, EW), jnp.int32),
            pltpu.VMEM((NWIN, EW), jnp.int32),
            pltpu.VMEM((EW, DP), jnp.float32),
            pltpu.VMEM_SHARED((NROWS, DP), jnp.float32),
            pltpu.SemaphoreType.DMA,
        ],
    )
    def k(g_hbm, src_hbm, dst_hbm, zeros_hbm, out_hbm, src_v, dst_v, rows_v, acc_sh, sem):
        cid = lax.axis_index("c")
        sid = lax.axis_index("s")
        wid = cid * NS + sid
        rows = NROWS // NS
        sl = pl.ds(sid * rows, rows)
        pltpu.async_copy(zeros_hbm.at[sl], acc_sh.at[sl], sem).wait()
        pltpu.async_copy(src_hbm.at[wid], src_v, sem).wait()
        pltpu.async_copy(dst_hbm.at[wid], dst_v, sem).wait()
        plsc.subcore_barrier()

        @pl.loop(0, NWIN)
        def _(w):
            pltpu.async_copy(g_hbm.at[src_v.at[w]], rows_v, sem).wait()
            pltpu.sync_copy(rows_v, acc_sh.at[dst_v.at[w]], add=True)

        plsc.subcore_barrier()
        pltpu.async_copy(acc_sh.at[sl], out_hbm.at[cid, sl], sem).wait()

    return k(g, src3, dst3, zeros)


def _sc_slot_scatter(featp, slots3, zeros_dense):
    @functools.partial(
        pl.kernel,
        out_type=jax.ShapeDtypeStruct((NC, TRASH, DP), jnp.float32),
        mesh=_sc_mesh(),
        scratch_types=[
            pltpu.VMEM((NODE_WIN, EW), jnp.int32),
            pltpu.VMEM((EW, DP), jnp.float32),
            pltpu.VMEM_SHARED((DENSE_PAD, DP), jnp.float32),
            pltpu.SemaphoreType.DMA,
        ],
    )
    def k(feat_hbm, slot_hbm, zeros_hbm, out_hbm, slot_v, rows_v, acc_sh, sem):
        cid = lax.axis_index("c")
        sid = lax.axis_index("s")
        wid = cid * NS + sid
        zrows = DENSE_PAD // NS
        zsl = pl.ds(sid * zrows, zrows)
        pltpu.async_copy(zeros_hbm.at[zsl], acc_sh.at[zsl], sem).wait()
        pltpu.async_copy(slot_hbm.at[wid], slot_v, sem).wait()
        plsc.subcore_barrier()
        base = wid * (NODE_WIN * EW)

        @pl.loop(0, NODE_WIN)
        def _(w):
            pltpu.async_copy(feat_hbm.at[pl.ds(base + w * EW, EW)], rows_v, sem).wait()
            pltpu.sync_copy(rows_v, acc_sh.at[slot_v.at[w]], add=True)

        plsc.subcore_barrier()
        orows = TRASH // NS
        osl = pl.ds(sid * orows, orows)
        pltpu.async_copy(acc_sh.at[osl], out_hbm.at[cid, osl], sem).wait()

    return k(featp, slots3, zeros_dense)


_R = 1000


def _tc_prep(deg_p, x, gw0p):
    def body(degp_ref, x_ref, w_ref, dinv_ref, g_ref):
        deg = degp_ref[0, :, 0:1] + degp_ref[1, :, 0:1] + 1.0
        dinv = 1.0 / jnp.sqrt(deg)
        dinv_ref[...] = dinv
        g_ref[...] = _dot(x_ref[...], w_ref[...]) * dinv

    return pl.pallas_call(
        body,
        grid=(N_NODES // _R,),
        in_specs=[
            pl.BlockSpec((2, _R, DP), lambda i: (0, i, 0)),
            pl.BlockSpec((_R, D_FEAT), lambda i: (i, 0)),
            pl.BlockSpec((D_FEAT, DP), lambda i: (0, 0)),
        ],
        out_specs=[
            pl.BlockSpec((_R, 1), lambda i: (i, 0)),
            pl.BlockSpec((_R, DP), lambda i: (i, 0)),
        ],
        out_shape=[
            jax.ShapeDtypeStruct((N_NODES, 1), jnp.float32),
            jax.ShapeDtypeStruct((N_NODES, DP), jnp.float32),
        ],
    )(deg_p, x, gw0p)


def _tc_layer(p, g, dinv, b, w):
    def body(p_ref, g_ref, dinv_ref, b_ref, w_ref, h_ref, gn_ref):
        dinv = dinv_ref[...]
        h = jnp.tanh(dinv * (p_ref[0] + p_ref[1] + g_ref[...]) + b_ref[...])
        h_ref[...] = h
        gn_ref[...] = _dot(h, w_ref[...]) * dinv

    return pl.pallas_call(
        body,
        grid=(N_NODES // _R,),
        in_specs=[
            pl.BlockSpec((2, _R, DP), lambda i: (0, i, 0)),
            pl.BlockSpec((_R, DP), lambda i: (i, 0)),
            pl.BlockSpec((_R, 1), lambda i: (i, 0)),
            pl.BlockSpec((1, DP), lambda i: (0, 0)),
            pl.BlockSpec((DP, DP), lambda i: (0, 0)),
        ],
        out_specs=[
            pl.BlockSpec((_R, DP), lambda i: (i, 0)),
            pl.BlockSpec((_R, DP), lambda i: (i, 0)),
        ],
        out_shape=[
            jax.ShapeDtypeStruct((N_NODES, DP), jnp.float32),
            jax.ShapeDtypeStruct((N_NODES, DP), jnp.float32),
        ],
    )(p, g, dinv, b, w)


def _tc_layer_last(p, g, dinv, b):
    def body(p_ref, g_ref, dinv_ref, b_ref, h_ref):
        dinv = dinv_ref[...]
        h_ref[...] = jnp.tanh(dinv * (p_ref[0] + p_ref[1] + g_ref[...]) + b_ref[...])

    return pl.pallas_call(
        body,
        grid=(N_NODES // _R,),
        in_specs=[
            pl.BlockSpec((2, _R, DP), lambda i: (0, i, 0)),
            pl.BlockSpec((_R, DP), lambda i: (i, 0)),
            pl.BlockSpec((_R, 1), lambda i: (i, 0)),
            pl.BlockSpec((1, DP), lambda i: (0, 0)),
        ],
        out_specs=pl.BlockSpec((_R, DP), lambda i: (i, 0)),
        out_shape=jax.ShapeDtypeStruct((N_NODES, DP), jnp.float32),
    )(p, g, dinv, b)


_BI = 512
_BJ = 2048


def _tc_slots(valc, valr, batc, batr):
    ni, nj = NPAD // _BI, NPAD // _BJ

    def body(vc_ref, vr_ref, bc_ref, br_ref, out_ref):
        i = pl.program_id(0)
        j = pl.program_id(1)

        @pl.when(j == 0)
        def _():
            out_ref[...] = jnp.zeros((_BI, 1), jnp.int32)

        bi = bc_ref[...]
        bj = br_ref[...]
        overlap = jnp.logical_and(jnp.max(bj) >= jnp.min(bi), jnp.min(bj) <= jnp.max(bi))

        @pl.when(overlap)
        def _():
            vi = vc_ref[...]
            vj = vr_ref[...]
            ii = i * _BI + lax.broadcasted_iota(jnp.int32, (_BI, 1), 0)
            jj = j * _BJ + lax.broadcasted_iota(jnp.int32, (1, _BJ), 1)
            beats = (vj > vi) | ((vj == vi) & (jj < ii))
            m = (bi == bj) & beats
            out_ref[...] += jnp.sum(m.astype(jnp.int32), axis=1, keepdims=True)

        @pl.when(j == nj - 1)
        def _():
            r = out_ref[...]
            keep = (r < K_SORT) & (bi < N_GRAPHS)
            out_ref[...] = jnp.where(keep, bi * K_SORT + r, TRASH)

    return pl.pallas_call(
        body,
        grid=(ni, nj),
        in_specs=[
            pl.BlockSpec((_BI, 1), lambda i, j: (i, 0)),
            pl.BlockSpec((1, _BJ), lambda i, j: (0, j)),
            pl.BlockSpec((_BI, 1), lambda i, j: (i, 0)),
            pl.BlockSpec((1, _BJ), lambda i, j: (0, j)),
        ],
        out_specs=pl.BlockSpec((_BI, 1), lambda i, j: (i, 0)),
        out_shape=jax.ShapeDtypeStruct((NPAD, 1), jnp.int32),
    )(valc, valr, batc, batr)


def _tc_readout(dense_p, c1mT, c1b, w2fT, c2b, mw1rT, mb1, mw2T, mb2):
    def body(dp_ref, c1_ref, c1b_ref, w2_ref, c2b_ref, m1_ref, m1b_ref, m2_ref, m2b_ref, out_ref):
        pooled = dp_ref[0] + dp_ref[1]
        a = pooled[:, 0:97]
        z1 = jax.nn.relu(_dot(a, c1_ref[...]) + c1b_ref[...])
        e = z1.reshape(N_GRAPHS * K_SORT // 2, 2, 16)
        zp = jnp.maximum(e[:, 0, :], e[:, 1, :])
        zp = zp.reshape(N_GRAPHS, K_SORT // 2, 16)
        outs = []
        for t in range(11):
            patch = jnp.concatenate([zp[:, t + dt, :] for dt in range(5)], axis=1)
            outs.append(jax.nn.relu(_dot(patch, w2_ref[...]) + c2b_ref[...]))
        flat = jnp.concatenate(outs, axis=1)
        z4 = jax.nn.relu(_dot(flat, m1_ref[...]) + m1b_ref[...])
        out_ref[...] = _dot(z4, m2_ref[...]) + m2b_ref[...]

    return pl.pallas_call(
        body,
        out_shape=jax.ShapeDtypeStruct((N_GRAPHS, 1), jnp.float32),
    )(dense_p, c1mT, c1b, w2fT, c2b, mw1rT, mb1, mw2T, mb2)


def kernel(x, edge_index, batch, gw0, gb0, gw1, gb1, gw2, gb2, gw3, gb3,
           c1w, c1b, c2w, c2b, mw1, mb1, mw2, mb2):
    src3 = edge_index[0].reshape(NW, NWIN, EW)
    dst3 = edge_index[1].reshape(NW, NWIN, EW)

    ones = jnp.ones((EW, DP), jnp.float32)
    zeros128 = jnp.zeros((NROWS, DP), jnp.float32)
    zeros_dense = jnp.zeros((DENSE_PAD, DP), jnp.float32)

    gw0p = jnp.pad(gw0, ((0, 0), (0, DP - HID)))
    gw1p = jnp.pad(gw1, ((0, DP - HID), (0, DP - HID)))
    gw2p = jnp.pad(gw2, ((0, DP - HID), (0, DP - HID)))
    gw3p = jnp.pad(gw3, ((0, DP - HID), (0, DP - 1)))
    gb0p = jnp.pad(gb0, (0, DP - HID)).reshape(1, DP)
    gb1p = jnp.pad(gb1, (0, DP - HID)).reshape(1, DP)
    gb2p = jnp.pad(gb2, (0, DP - HID)).reshape(1, DP)
    gb3p = jnp.pad(gb3, (0, DP - 1)).reshape(1, DP)

    deg_p = _sc_deg(dst3, ones, zeros128)
    dinv, g1 = _tc_prep(deg_p, x, gw0p)

    p1 = _sc_seg(g1, src3, dst3, zeros128)
    h1, g2 = _tc_layer(p1, g1, dinv, gb0p, gw1p)
    p2 = _sc_seg(g2, src3, dst3, zeros128)
    h2, g3 = _tc_layer(p2, g2, dinv, gb1p, gw2p)
    p3 = _sc_seg(g3, src3, dst3, zeros128)
    h3, g4 = _tc_layer(p3, g3, dinv, gb2p, gw3p)
    p4 = _sc_seg(g4, src3, dst3, zeros128)
    h4 = _tc_layer_last(p4, g4, dinv, gb3p)

    h4col = h4[:, 0:1]
    feat = jnp.concatenate([h1[:, :HID], h2[:, :HID], h3[:, :HID], h4col], axis=1)
    featp = jnp.pad(feat, ((0, NPAD - N_NODES), (0, DP - 97)))

    valp = jnp.pad(h4col[:, 0], (0, NPAD - N_NODES))
    batp = jnp.pad(batch, (0, NPAD - N_NODES), constant_values=N_GRAPHS)
    slots = _tc_slots(valp.reshape(NPAD, 1), valp.reshape(1, NPAD),
                      batp.reshape(NPAD, 1), batp.reshape(1, NPAD))
    slots3 = slots.reshape(NW, NODE_WIN, EW)

    dense_p = _sc_slot_scatter(featp, slots3, zeros_dense)

    c1mT = c1w[:, 0, :].T
    w2fT = c2w.transpose(0, 2, 1).reshape(32, 80).T
    mw1rT = mw1.reshape(128, 32, 11).transpose(0, 2, 1).reshape(128, 352).T
    return _tc_readout(dense_p, c1mT, c1b.reshape(1, 16), w2fT,
                       c2b.reshape(1, 32), mw1rT, mb1.reshape(1, 128),
                       mw2.T, mb2.reshape(1, 1))
